# baseline (device time: 85493 ns/iter reference)
import jax
import jax.numpy as jnp
from jax import lax
from jax.experimental import pallas as pl
from jax.experimental.pallas import tpu as pltpu

N_DEV = 4


def kernel(x, router_W, route_idx, expert_W):
    n_tok, d = x.shape
    n_exp = router_W.shape[1]
    e_loc, _, h = expert_W.shape

    def body(x_ref, rw_ref, idx_ref, ew_ref, out_ref, comm_ref, send_sems, recv_sems):
        my = lax.axis_index("i")
        left = lax.rem(my + N_DEV - 1, N_DEV)
        right = lax.rem(my + 1, N_DEV)

        barrier_sem = pltpu.get_barrier_semaphore()
        for nbr in (left, right):
            pl.semaphore_signal(
                barrier_sem, inc=1,
                device_id=(nbr,), device_id_type=pl.DeviceIdType.MESH,
            )
        pl.semaphore_wait(barrier_sem, 2)

        xv = x_ref[:, :]
        scores = jnp.dot(xv, rw_ref[:, :], preferred_element_type=jnp.float32)
        s_max = jnp.max(scores, axis=-1, keepdims=True)
        probs = jnp.exp(scores - s_max)
        probs = probs / jnp.sum(probs, axis=-1, keepdims=True)

        idx = idx_ref[:, :]
        eidx = lax.broadcasted_iota(jnp.int32, (n_tok, n_exp), 1)
        mask = (idx[:, 0:1] == eidx) | (idx[:, 1:2] == eidx)
        sel = jnp.where(mask, probs, 0.0)
        gates = sel / jnp.sum(sel, axis=-1, keepdims=True)

        base = my * e_loc
        acc = jnp.zeros((n_tok, h), jnp.float32)
        for k in range(e_loc):
            gk = jnp.sum(
                jnp.where(eidx == base + k, gates, 0.0), axis=-1, keepdims=True
            )
            acc = acc + jnp.dot(
                xv * gk, ew_ref[k], preferred_element_type=jnp.float32
            )

        out_ref[:, :] = acc
        comm_ref[0] = acc

        for hop in range(N_DEV - 1):
            rdma = pltpu.make_async_remote_copy(
                src_ref=comm_ref.at[hop],
                dst_ref=comm_ref.at[hop + 1],
                send_sem=send_sems.at[hop],
                recv_sem=recv_sems.at[hop],
                device_id=(right,),
                device_id_type=pl.DeviceIdType.MESH,
            )
            rdma.start()
            rdma.wait()
            out_ref[:, :] += comm_ref[hop + 1]

    return pl.pallas_call(
        body,
        out_shape=jax.ShapeDtypeStruct((n_tok, h), jnp.float32),
        in_specs=[
            pl.BlockSpec(memory_space=pltpu.VMEM),
            pl.BlockSpec(memory_space=pltpu.VMEM),
            pl.BlockSpec(memory_space=pltpu.VMEM),
            pl.BlockSpec(memory_space=pltpu.VMEM),
        ],
        out_specs=pl.BlockSpec(memory_space=pltpu.VMEM),
        scratch_shapes=[
            pltpu.VMEM((N_DEV, n_tok, h), jnp.float32),
            pltpu.SemaphoreType.DMA((N_DEV - 1,)),
            pltpu.SemaphoreType.DMA((N_DEV - 1,)),
        ],
        compiler_params=pltpu.CompilerParams(collective_id=0),
    )(x, router_W, route_idx, expert_W)


# device time: 39124 ns/iter; 2.1852x vs baseline; 2.1852x over previous
import jax
import jax.numpy as jnp
from jax import lax
from jax.experimental import pallas as pl
from jax.experimental.pallas import tpu as pltpu

N_DEV = 4


def kernel(x, router_W, route_idx, expert_W):
    n_tok, d = x.shape
    n_exp = router_W.shape[1]
    e_loc, _, h = expert_W.shape
    ch = n_tok // N_DEV

    def body(x_ref, rw_ref, idx_ref, ew_ref, out_ref,
             part_ref, rsbuf_ref, red_ref, rs_send, rs_recv, ag_send, ag_recv):
        my = lax.axis_index("i")

        barrier_sem = pltpu.get_barrier_semaphore()
        for k in range(1, N_DEV):
            pl.semaphore_signal(
                barrier_sem, inc=1,
                device_id=(lax.rem(my + k, N_DEV),),
                device_id_type=pl.DeviceIdType.MESH,
            )
        pl.semaphore_wait(barrier_sem, N_DEV - 1)

        xv = x_ref[:, :]
        scores = jnp.dot(xv, rw_ref[:, :], preferred_element_type=jnp.float32)
        s_max = jnp.max(scores, axis=-1, keepdims=True)
        probs = jnp.exp(scores - s_max)
        probs = probs / jnp.sum(probs, axis=-1, keepdims=True)

        idx = idx_ref[:, :]
        eidx = lax.broadcasted_iota(jnp.int32, (n_tok, n_exp), 1)
        mask = (idx[:, 0:1] == eidx) | (idx[:, 1:2] == eidx)
        sel = jnp.where(mask, probs, 0.0)
        gates = sel / jnp.sum(sel, axis=-1, keepdims=True)

        base = my * e_loc
        acc = jnp.zeros((n_tok, h), jnp.float32)
        for k in range(e_loc):
            gk = jnp.sum(
                jnp.where(eidx == base + k, gates, 0.0), axis=-1, keepdims=True
            )
            acc = acc + jnp.dot(
                xv * gk, ew_ref[k], preferred_element_type=jnp.float32
            )
        part_ref[:, :] = acc

        rs_sends = []
        for k in range(1, N_DEV):
            tgt = lax.rem(my + k, N_DEV)
            rdma = pltpu.make_async_remote_copy(
                src_ref=part_ref.at[pl.ds(tgt * ch, ch), :],
                dst_ref=rsbuf_ref.at[3 - k],
                send_sem=rs_send.at[k - 1],
                recv_sem=rs_recv.at[3 - k],
                device_id=(tgt,),
                device_id_type=pl.DeviceIdType.MESH,
            )
            rdma.start()
            rs_sends.append(rdma)

        for slot in range(N_DEV - 1):
            pltpu.make_async_remote_copy(
                src_ref=rsbuf_ref.at[slot],
                dst_ref=rsbuf_ref.at[slot],
                send_sem=rs_send.at[0],
                recv_sem=rs_recv.at[slot],
                device_id=(my,),
                device_id_type=pl.DeviceIdType.MESH,
            ).wait_recv()

        red = (
            part_ref[pl.ds(my * ch, ch), :]
            + rsbuf_ref[0] + rsbuf_ref[1] + rsbuf_ref[2]
        )
        red_ref[:, :] = red

        ag_sends = []
        for k in range(1, N_DEV):
            tgt = lax.rem(my + k, N_DEV)
            rdma = pltpu.make_async_remote_copy(
                src_ref=red_ref,
                dst_ref=out_ref.at[pl.ds(my * ch, ch), :],
                send_sem=ag_send.at[k - 1],
                recv_sem=ag_recv.at[3 - k],
                device_id=(tgt,),
                device_id_type=pl.DeviceIdType.MESH,
            )
            rdma.start()
            ag_sends.append(rdma)

        out_ref[pl.ds(my * ch, ch), :] = red

        for slot in range(N_DEV - 1):
            pltpu.make_async_remote_copy(
                src_ref=red_ref,
                dst_ref=out_ref.at[pl.ds(my * ch, ch), :],
                send_sem=ag_send.at[0],
                recv_sem=ag_recv.at[slot],
                device_id=(my,),
                device_id_type=pl.DeviceIdType.MESH,
            ).wait_recv()

        for rdma in rs_sends + ag_sends:
            rdma.wait_send()

    return pl.pallas_call(
        body,
        out_shape=jax.ShapeDtypeStruct((n_tok, h), jnp.float32),
        in_specs=[
            pl.BlockSpec(memory_space=pltpu.VMEM),
            pl.BlockSpec(memory_space=pltpu.VMEM),
            pl.BlockSpec(memory_space=pltpu.VMEM),
            pl.BlockSpec(memory_space=pltpu.VMEM),
        ],
        out_specs=pl.BlockSpec(memory_space=pltpu.VMEM),
        scratch_shapes=[
            pltpu.VMEM((n_tok, h), jnp.float32),
            pltpu.VMEM((N_DEV - 1, ch, h), jnp.float32),
            pltpu.VMEM((ch, h), jnp.float32),
            pltpu.SemaphoreType.DMA((N_DEV - 1,)),
            pltpu.SemaphoreType.DMA((N_DEV - 1,)),
            pltpu.SemaphoreType.DMA((N_DEV - 1,)),
            pltpu.SemaphoreType.DMA((N_DEV - 1,)),
        ],
        compiler_params=pltpu.CompilerParams(collective_id=0),
    )(x, router_W, route_idx, expert_W)


# device time: 37133 ns/iter; 2.3023x vs baseline; 1.0536x over previous
import jax
import jax.numpy as jnp
from jax import lax
from jax.experimental import pallas as pl
from jax.experimental.pallas import tpu as pltpu

N_DEV = 4


def kernel(x, router_W, route_idx, expert_W):
    n_tok, d = x.shape
    n_exp = router_W.shape[1]
    e_loc, _, h = expert_W.shape
    ch = n_tok // N_DEV

    def body(x_ref, rw_ref, idx_ref, ew_ref, out_ref,
             part_ref, rsbuf_ref, red_ref, rs_send, rs_recv, ag_send, ag_recv):
        my = lax.axis_index("i")

        barrier_sem = pltpu.get_barrier_semaphore()
        for k in range(1, N_DEV):
            pl.semaphore_signal(
                barrier_sem, inc=1,
                device_id=(lax.rem(my + k, N_DEV),),
                device_id_type=pl.DeviceIdType.MESH,
            )
        pl.semaphore_wait(barrier_sem, N_DEV - 1)

        base = my * e_loc
        eidx = lax.broadcasted_iota(jnp.int32, (ch, n_exp), 1)

        def partial_chunk(c):
            rows = pl.ds(c * ch, ch)
            xb = x_ref[rows, :]
            scores = jnp.dot(xb, rw_ref[:, :], preferred_element_type=jnp.float32)
            s_max = jnp.max(scores, axis=-1, keepdims=True)
            probs = jnp.exp(scores - s_max)
            probs = probs / jnp.sum(probs, axis=-1, keepdims=True)
            idx = idx_ref[rows, :]
            mask = (idx[:, 0:1] == eidx) | (idx[:, 1:2] == eidx)
            sel = jnp.where(mask, probs, 0.0)
            gates = sel / jnp.sum(sel, axis=-1, keepdims=True)
            acc = jnp.zeros((ch, h), jnp.float32)
            for k in range(e_loc):
                gk = jnp.sum(
                    jnp.where(eidx == base + k, gates, 0.0), axis=-1, keepdims=True
                )
                acc = acc + jnp.dot(
                    xb * gk, ew_ref[k], preferred_element_type=jnp.float32
                )
            return acc

        rs_sends = []
        for k in range(1, N_DEV):
            tgt = lax.rem(my + k, N_DEV)
            part_ref[pl.ds(tgt * ch, ch), :] = partial_chunk(tgt)
            rdma = pltpu.make_async_remote_copy(
                src_ref=part_ref.at[pl.ds(tgt * ch, ch), :],
                dst_ref=rsbuf_ref.at[3 - k],
                send_sem=rs_send.at[k - 1],
                recv_sem=rs_recv.at[3 - k],
                device_id=(tgt,),
                device_id_type=pl.DeviceIdType.MESH,
            )
            rdma.start()
            rs_sends.append(rdma)
        part_ref[pl.ds(my * ch, ch), :] = partial_chunk(my)

        for slot in range(N_DEV - 1):
            pltpu.make_async_remote_copy(
                src_ref=rsbuf_ref.at[slot],
                dst_ref=rsbuf_ref.at[slot],
                send_sem=rs_send.at[0],
                recv_sem=rs_recv.at[slot],
                device_id=(my,),
                device_id_type=pl.DeviceIdType.MESH,
            ).wait_recv()

        red = (
            part_ref[pl.ds(my * ch, ch), :]
            + rsbuf_ref[0] + rsbuf_ref[1] + rsbuf_ref[2]
        )
        red_ref[:, :] = red

        ag_sends = []
        for k in range(1, N_DEV):
            tgt = lax.rem(my + k, N_DEV)
            rdma = pltpu.make_async_remote_copy(
                src_ref=red_ref,
                dst_ref=out_ref.at[pl.ds(my * ch, ch), :],
                send_sem=ag_send.at[k - 1],
                recv_sem=ag_recv.at[3 - k],
                device_id=(tgt,),
                device_id_type=pl.DeviceIdType.MESH,
            )
            rdma.start()
            ag_sends.append(rdma)

        out_ref[pl.ds(my * ch, ch), :] = red

        for slot in range(N_DEV - 1):
            pltpu.make_async_remote_copy(
                src_ref=red_ref,
                dst_ref=out_ref.at[pl.ds(my * ch, ch), :],
                send_sem=ag_send.at[0],
                recv_sem=ag_recv.at[slot],
                device_id=(my,),
                device_id_type=pl.DeviceIdType.MESH,
            ).wait_recv()

        for rdma in rs_sends + ag_sends:
            rdma.wait_send()

    return pl.pallas_call(
        body,
        out_shape=jax.ShapeDtypeStruct((n_tok, h), jnp.float32),
        in_specs=[
            pl.BlockSpec(memory_space=pltpu.VMEM),
            pl.BlockSpec(memory_space=pltpu.VMEM),
            pl.BlockSpec(memory_space=pltpu.VMEM),
            pl.BlockSpec(memory_space=pltpu.VMEM),
        ],
        out_specs=pl.BlockSpec(memory_space=pltpu.VMEM),
        scratch_shapes=[
            pltpu.VMEM((n_tok, h), jnp.float32),
            pltpu.VMEM((N_DEV - 1, ch, h), jnp.float32),
            pltpu.VMEM((ch, h), jnp.float32),
            pltpu.SemaphoreType.DMA((N_DEV - 1,)),
            pltpu.SemaphoreType.DMA((N_DEV - 1,)),
            pltpu.SemaphoreType.DMA((N_DEV - 1,)),
            pltpu.SemaphoreType.DMA((N_DEV - 1,)),
        ],
        compiler_params=pltpu.CompilerParams(collective_id=0),
    )(x, router_W, route_idx, expert_W)


# device time: 25979 ns/iter; 3.2909x vs baseline; 1.4293x over previous
import jax
import jax.numpy as jnp
from jax import lax
from jax.experimental import pallas as pl
from jax.experimental.pallas import tpu as pltpu

N_DEV = 4


def kernel(x, router_W, route_idx, expert_W):
    n_tok, d = x.shape
    n_exp = router_W.shape[1]
    e_loc, _, h = expert_W.shape
    ch = n_tok // N_DEV

    def body(x_ref, rw_ref, idx_ref, ew_ref, out_ref,
             rs_sbuf, rs_rbuf, red_sref, ag_rbuf,
             rs_send, rs_recv, ag_send, ag_recv):
        my = lax.axis_index("i")

        barrier_sem = pltpu.get_barrier_semaphore()
        for k in range(1, N_DEV):
            pl.semaphore_signal(
                barrier_sem, inc=1,
                device_id=(lax.rem(my + k, N_DEV),),
                device_id_type=pl.DeviceIdType.MESH,
            )
        pl.semaphore_wait(barrier_sem, N_DEV - 1)

        base = my * e_loc
        eidx = lax.broadcasted_iota(jnp.int32, (ch, n_exp), 1)

        def partial_chunk(c):
            rows = pl.ds(c * ch, ch)
            xb = x_ref[rows, :]
            scores = jnp.dot(xb, rw_ref[:, :], preferred_element_type=jnp.float32)
            s_max = jnp.max(scores, axis=-1, keepdims=True)
            probs = jnp.exp(scores - s_max)
            probs = probs / jnp.sum(probs, axis=-1, keepdims=True)
            idx = idx_ref[rows, :]
            mask = (idx[:, 0:1] == eidx) | (idx[:, 1:2] == eidx)
            sel = jnp.where(mask, probs, 0.0)
            gates = sel / jnp.sum(sel, axis=-1, keepdims=True)
            acc = jnp.zeros((ch, h), jnp.float32)
            for k in range(e_loc):
                gk = jnp.sum(
                    jnp.where(eidx == base + k, gates, 0.0), axis=-1, keepdims=True
                )
                acc = acc + jnp.dot(
                    xb * gk, ew_ref[k], preferred_element_type=jnp.float32
                )
            return acc

        rs_sends = []
        for k in range(1, N_DEV):
            tgt = lax.rem(my + k, N_DEV)
            rs_sbuf[k - 1] = partial_chunk(tgt).astype(jnp.bfloat16)
            rdma = pltpu.make_async_remote_copy(
                src_ref=rs_sbuf.at[k - 1],
                dst_ref=rs_rbuf.at[3 - k],
                send_sem=rs_send.at[k - 1],
                recv_sem=rs_recv.at[3 - k],
                device_id=(tgt,),
                device_id_type=pl.DeviceIdType.MESH,
            )
            rdma.start()
            rs_sends.append(rdma)
        own = partial_chunk(my)

        for slot in range(N_DEV - 1):
            pltpu.make_async_remote_copy(
                src_ref=rs_rbuf.at[slot],
                dst_ref=rs_rbuf.at[slot],
                send_sem=rs_send.at[0],
                recv_sem=rs_recv.at[slot],
                device_id=(my,),
                device_id_type=pl.DeviceIdType.MESH,
            ).wait_recv()

        red = (
            own
            + rs_rbuf[0].astype(jnp.float32)
            + rs_rbuf[1].astype(jnp.float32)
            + rs_rbuf[2].astype(jnp.float32)
        )
        red_sref[:, :] = red.astype(jnp.bfloat16)

        ag_sends = []
        for k in range(1, N_DEV):
            tgt = lax.rem(my + k, N_DEV)
            rdma = pltpu.make_async_remote_copy(
                src_ref=red_sref,
                dst_ref=ag_rbuf.at[3 - k],
                send_sem=ag_send.at[k - 1],
                recv_sem=ag_recv.at[3 - k],
                device_id=(tgt,),
                device_id_type=pl.DeviceIdType.MESH,
            )
            rdma.start()
            ag_sends.append(rdma)

        out_ref[pl.ds(my * ch, ch), :] = red

        for m in range(N_DEV - 1):
            pltpu.make_async_remote_copy(
                src_ref=ag_rbuf.at[m],
                dst_ref=ag_rbuf.at[m],
                send_sem=ag_send.at[0],
                recv_sem=ag_recv.at[m],
                device_id=(my,),
                device_id_type=pl.DeviceIdType.MESH,
            ).wait_recv()
            s = lax.rem(my + m + 1, N_DEV)
            out_ref[pl.ds(s * ch, ch), :] = ag_rbuf[m].astype(jnp.float32)

        for rdma in rs_sends + ag_sends:
            rdma.wait_send()

    return pl.pallas_call(
        body,
        out_shape=jax.ShapeDtypeStruct((n_tok, h), jnp.float32),
        in_specs=[
            pl.BlockSpec(memory_space=pltpu.VMEM),
            pl.BlockSpec(memory_space=pltpu.VMEM),
            pl.BlockSpec(memory_space=pltpu.VMEM),
            pl.BlockSpec(memory_space=pltpu.VMEM),
        ],
        out_specs=pl.BlockSpec(memory_space=pltpu.VMEM),
        scratch_shapes=[
            pltpu.VMEM((N_DEV - 1, ch, h), jnp.bfloat16),
            pltpu.VMEM((N_DEV - 1, ch, h), jnp.bfloat16),
            pltpu.VMEM((ch, h), jnp.bfloat16),
            pltpu.VMEM((N_DEV - 1, ch, h), jnp.bfloat16),
            pltpu.SemaphoreType.DMA((N_DEV - 1,)),
            pltpu.SemaphoreType.DMA((N_DEV - 1,)),
            pltpu.SemaphoreType.DMA((N_DEV - 1,)),
            pltpu.SemaphoreType.DMA((N_DEV - 1,)),
        ],
        compiler_params=pltpu.CompilerParams(collective_id=0),
    )(x, router_W, route_idx, expert_W)


# device time: 25862 ns/iter; 3.3057x vs baseline; 1.0045x over previous
import jax
import jax.numpy as jnp
from jax import lax
from jax.experimental import pallas as pl
from jax.experimental.pallas import tpu as pltpu

N_DEV = 4


def kernel(x, router_W, route_idx, expert_W):
    n_tok, d = x.shape
    n_exp = router_W.shape[1]
    e_loc, _, h = expert_W.shape
    ch = n_tok // N_DEV

    def body(x_ref, rw_ref, idx_ref, ew_ref, out_ref,
             ew_bf, rs_sbuf, rs_rbuf, red_sref, ag_rbuf,
             rs_send, rs_recv, ag_send, ag_recv):
        my = lax.axis_index("i")

        barrier_sem = pltpu.get_barrier_semaphore()
        for k in range(1, N_DEV):
            pl.semaphore_signal(
                barrier_sem, inc=1,
                device_id=(lax.rem(my + k, N_DEV),),
                device_id_type=pl.DeviceIdType.MESH,
            )
        pl.semaphore_wait(barrier_sem, N_DEV - 1)

        base = my * e_loc
        eidx = lax.broadcasted_iota(jnp.int32, (ch, n_exp), 1)
        ew_bf[:, :, :] = ew_ref[:, :, :].astype(jnp.bfloat16)

        def partial_chunk(c):
            rows = pl.ds(c * ch, ch)
            xb = x_ref[rows, :]
            scores = jnp.dot(xb, rw_ref[:, :], preferred_element_type=jnp.float32)
            s_max = jnp.max(scores, axis=-1, keepdims=True)
            probs = jnp.exp(scores - s_max)
            probs = probs / jnp.sum(probs, axis=-1, keepdims=True)
            idx = idx_ref[rows, :]
            mask = (idx[:, 0:1] == eidx) | (idx[:, 1:2] == eidx)
            sel = jnp.where(mask, probs, 0.0)
            gates = sel / jnp.sum(sel, axis=-1, keepdims=True)
            acc = jnp.zeros((ch, h), jnp.float32)
            for k in range(e_loc):
                gk = jnp.sum(
                    jnp.where(eidx == base + k, gates, 0.0), axis=-1, keepdims=True
                )
                acc = acc + jnp.dot(
                    (xb * gk).astype(jnp.bfloat16), ew_bf[k],
                    preferred_element_type=jnp.float32,
                )
            return acc

        rs_sends = []
        for k in range(1, N_DEV):
            tgt = lax.rem(my + k, N_DEV)
            rs_sbuf[k - 1] = partial_chunk(tgt).astype(jnp.bfloat16)
            rdma = pltpu.make_async_remote_copy(
                src_ref=rs_sbuf.at[k - 1],
                dst_ref=rs_rbuf.at[3 - k],
                send_sem=rs_send.at[k - 1],
                recv_sem=rs_recv.at[3 - k],
                device_id=(tgt,),
                device_id_type=pl.DeviceIdType.MESH,
            )
            rdma.start()
            rs_sends.append(rdma)
        own = partial_chunk(my)

        for slot in range(N_DEV - 1):
            pltpu.make_async_remote_copy(
                src_ref=rs_rbuf.at[slot],
                dst_ref=rs_rbuf.at[slot],
                send_sem=rs_send.at[0],
                recv_sem=rs_recv.at[slot],
                device_id=(my,),
                device_id_type=pl.DeviceIdType.MESH,
            ).wait_recv()

        red = (
            own
            + rs_rbuf[0].astype(jnp.float32)
            + rs_rbuf[1].astype(jnp.float32)
            + rs_rbuf[2].astype(jnp.float32)
        )
        red_sref[:, :] = red.astype(jnp.bfloat16)

        ag_sends = []
        for k in range(1, N_DEV):
            tgt = lax.rem(my + k, N_DEV)
            rdma = pltpu.make_async_remote_copy(
                src_ref=red_sref,
                dst_ref=ag_rbuf.at[3 - k],
                send_sem=ag_send.at[k - 1],
                recv_sem=ag_recv.at[3 - k],
                device_id=(tgt,),
                device_id_type=pl.DeviceIdType.MESH,
            )
            rdma.start()
            ag_sends.append(rdma)

        out_ref[pl.ds(my * ch, ch), :] = red

        for m in range(N_DEV - 1):
            pltpu.make_async_remote_copy(
                src_ref=ag_rbuf.at[m],
                dst_ref=ag_rbuf.at[m],
                send_sem=ag_send.at[0],
                recv_sem=ag_recv.at[m],
                device_id=(my,),
                device_id_type=pl.DeviceIdType.MESH,
            ).wait_recv()
            s = lax.rem(my + m + 1, N_DEV)
            out_ref[pl.ds(s * ch, ch), :] = ag_rbuf[m].astype(jnp.float32)

        for rdma in rs_sends + ag_sends:
            rdma.wait_send()

    return pl.pallas_call(
        body,
        out_shape=jax.ShapeDtypeStruct((n_tok, h), jnp.float32),
        in_specs=[
            pl.BlockSpec(memory_space=pltpu.VMEM),
            pl.BlockSpec(memory_space=pltpu.VMEM),
            pl.BlockSpec(memory_space=pltpu.VMEM),
            pl.BlockSpec(memory_space=pltpu.VMEM),
        ],
        out_specs=pl.BlockSpec(memory_space=pltpu.VMEM),
        scratch_shapes=[
            pltpu.VMEM((e_loc, d, h), jnp.bfloat16),
            pltpu.VMEM((N_DEV - 1, ch, h), jnp.bfloat16),
            pltpu.VMEM((N_DEV - 1, ch, h), jnp.bfloat16),
            pltpu.VMEM((ch, h), jnp.bfloat16),
            pltpu.VMEM((N_DEV - 1, ch, h), jnp.bfloat16),
            pltpu.SemaphoreType.DMA((N_DEV - 1,)),
            pltpu.SemaphoreType.DMA((N_DEV - 1,)),
            pltpu.SemaphoreType.DMA((N_DEV - 1,)),
            pltpu.SemaphoreType.DMA((N_DEV - 1,)),
        ],
        compiler_params=pltpu.CompilerParams(collective_id=0),
    )(x, router_W, route_idx, expert_W)


# device time: 25604 ns/iter; 3.3390x vs baseline; 1.0101x over previous
import jax
import jax.numpy as jnp
from jax import lax
from jax.experimental import pallas as pl
from jax.experimental.pallas import tpu as pltpu

N_DEV = 4


def kernel(x, router_W, route_idx, expert_W):
    n_tok, d = x.shape
    n_exp = router_W.shape[1]
    e_loc, _, h = expert_W.shape
    ch = n_tok // N_DEV

    def body(x_ref, rw_ref, idx_ref, ew_ref, out_ref,
             ew_bf, rs_sbuf, rs_rbuf, red_sref, ag_rbuf,
             rs_send, rs_recv, ag_send, ag_recv):
        my = lax.axis_index("i")

        barrier_sem = pltpu.get_barrier_semaphore()
        for k in range(1, N_DEV):
            pl.semaphore_signal(
                barrier_sem, inc=1,
                device_id=(lax.rem(my + k, N_DEV),),
                device_id_type=pl.DeviceIdType.MESH,
            )
        pl.semaphore_wait(barrier_sem, N_DEV - 1)

        base = my * e_loc
        eidx = lax.broadcasted_iota(jnp.int32, (ch, n_exp), 1)
        ew_bf[:, :, :] = ew_ref[:, :, :].astype(jnp.bfloat16)

        def partial_chunk(c):
            rows = pl.ds(c * ch, ch)
            xb = x_ref[rows, :]
            scores = jnp.dot(xb, rw_ref[:, :], preferred_element_type=jnp.float32)
            s_max = jnp.max(scores, axis=-1, keepdims=True)
            probs = jnp.exp(scores - s_max)
            probs = probs / jnp.sum(probs, axis=-1, keepdims=True)
            idx = idx_ref[rows, :]
            mask = (idx[:, 0:1] == eidx) | (idx[:, 1:2] == eidx)
            sel = jnp.where(mask, probs, 0.0)
            gates = sel / jnp.sum(sel, axis=-1, keepdims=True)
            acc = jnp.zeros((ch, h), jnp.float32)
            for k in range(e_loc):
                gk = jnp.sum(
                    jnp.where(eidx == base + k, gates, 0.0), axis=-1, keepdims=True
                )
                acc = acc + jnp.dot(
                    (xb * gk).astype(jnp.bfloat16), ew_bf[k],
                    preferred_element_type=jnp.float32,
                )
            return acc

        rs_sends = []
        for k in (2, 1, 3):
            tgt = lax.rem(my + k, N_DEV)
            rs_sbuf[k - 1] = partial_chunk(tgt).astype(jnp.bfloat16)
            rdma = pltpu.make_async_remote_copy(
                src_ref=rs_sbuf.at[k - 1],
                dst_ref=rs_rbuf.at[3 - k],
                send_sem=rs_send.at[k - 1],
                recv_sem=rs_recv.at[3 - k],
                device_id=(tgt,),
                device_id_type=pl.DeviceIdType.MESH,
            )
            rdma.start()
            rs_sends.append(rdma)
        own = partial_chunk(my)

        red = own
        for slot in range(N_DEV - 1):
            pltpu.make_async_remote_copy(
                src_ref=rs_rbuf.at[slot],
                dst_ref=rs_rbuf.at[slot],
                send_sem=rs_send.at[0],
                recv_sem=rs_recv.at[slot],
                device_id=(my,),
                device_id_type=pl.DeviceIdType.MESH,
            ).wait_recv()
            red = red + rs_rbuf[slot].astype(jnp.float32)

        red_sref[:, :] = red.astype(jnp.bfloat16)

        ag_sends = []
        for k in (2, 1, 3):
            tgt = lax.rem(my + k, N_DEV)
            rdma = pltpu.make_async_remote_copy(
                src_ref=red_sref,
                dst_ref=ag_rbuf.at[3 - k],
                send_sem=ag_send.at[k - 1],
                recv_sem=ag_recv.at[3 - k],
                device_id=(tgt,),
                device_id_type=pl.DeviceIdType.MESH,
            )
            rdma.start()
            ag_sends.append(rdma)

        out_ref[pl.ds(my * ch, ch), :] = red

        for m in range(N_DEV - 1):
            pltpu.make_async_remote_copy(
                src_ref=ag_rbuf.at[m],
                dst_ref=ag_rbuf.at[m],
                send_sem=ag_send.at[0],
                recv_sem=ag_recv.at[m],
                device_id=(my,),
                device_id_type=pl.DeviceIdType.MESH,
            ).wait_recv()
            s = lax.rem(my + m + 1, N_DEV)
            out_ref[pl.ds(s * ch, ch), :] = ag_rbuf[m].astype(jnp.float32)

        for rdma in rs_sends + ag_sends:
            rdma.wait_send()

    return pl.pallas_call(
        body,
        out_shape=jax.ShapeDtypeStruct((n_tok, h), jnp.float32),
        in_specs=[
            pl.BlockSpec(memory_space=pltpu.VMEM),
            pl.BlockSpec(memory_space=pltpu.VMEM),
            pl.BlockSpec(memory_space=pltpu.VMEM),
            pl.BlockSpec(memory_space=pltpu.VMEM),
        ],
        out_specs=pl.BlockSpec(memory_space=pltpu.VMEM),
        scratch_shapes=[
            pltpu.VMEM((e_loc, d, h), jnp.bfloat16),
            pltpu.VMEM((N_DEV - 1, ch, h), jnp.bfloat16),
            pltpu.VMEM((N_DEV - 1, ch, h), jnp.bfloat16),
            pltpu.VMEM((ch, h), jnp.bfloat16),
            pltpu.VMEM((N_DEV - 1, ch, h), jnp.bfloat16),
            pltpu.SemaphoreType.DMA((N_DEV - 1,)),
            pltpu.SemaphoreType.DMA((N_DEV - 1,)),
            pltpu.SemaphoreType.DMA((N_DEV - 1,)),
            pltpu.SemaphoreType.DMA((N_DEV - 1,)),
        ],
        compiler_params=pltpu.CompilerParams(collective_id=0),
    )(x, router_W, route_idx, expert_W)
